# baseline (speedup 1.0000x reference)
import jax
import jax.numpy as jnp
from jax.experimental import pallas as pl
from jax.experimental.pallas import tpu as pltpu

_N = 4096
_E = 8
_CAP = 640
_TEMP = 1.5
_NOISE_STD = 0.02
_NEG = -1


def _compute_body(x_ref, wg_ref, noise_ref,
                  lg_ref, ep_ref, ei_ref, pm_ref, z_ref, std_ref):
    x = x_ref[...]
    wg = wg_ref[...]
    logits = jnp.dot(x, wg, preferred_element_type=jnp.float32)
    lt = jnp.clip(logits / _TEMP, -10.0, 10.0)
    m = jnp.mean(lt)
    std_ref[...] = jnp.sqrt(jnp.mean((lt - m) ** 2, keepdims=True))
    row_max = jnp.max(lt, axis=1, keepdims=True)
    lse = row_max + jnp.log(jnp.sum(jnp.exp(lt - row_max), axis=1,
                                    keepdims=True))
    z_ref[...] = jnp.mean(lse * lse, keepdims=True)
    lg = lt + noise_ref[...]
    lg_ref[...] = lg
    mx = jnp.max(lg, axis=1, keepdims=True)
    unnorm = jnp.exp(lg - mx)
    s = jnp.sum(unnorm, axis=1, keepdims=True)
    probs = unnorm / s
    pm_ref[...] = jnp.mean(probs, axis=0, keepdims=True)
    epmax = jnp.max(probs, axis=1, keepdims=True)
    ep_ref[...] = epmax
    iota_e = jax.lax.broadcasted_iota(jnp.int32, (_N, _E), 1)
    ei_ref[...] = jnp.min(jnp.where(probs == epmax, iota_e, _E), axis=1,
                          keepdims=True)


def _rank_body(ep_col_ref, ei_col_ref, ep_row_ref, ei_row_ref, pm_ref,
               flat_ref, cnt_ref, aux_ref):
    i = pl.program_id(0)
    blk = ep_col_ref.shape[0]
    pn = ep_col_ref[...]
    en = ei_col_ref[...]
    n_idx = i * blk + jax.lax.broadcasted_iota(jnp.int32, (blk, 1), 0)
    cnt = jnp.zeros((blk, 1), jnp.int32)
    chunk = 1024
    for k in range(_N // chunk):
        pm_r = ep_row_ref[:, k * chunk:(k + 1) * chunk]
        em_r = ei_row_ref[:, k * chunk:(k + 1) * chunk]
        m_idx = k * chunk + jax.lax.broadcasted_iota(
            jnp.int32, (1, chunk), 1)
        ahead = (pm_r > pn) | ((pm_r == pn) & (m_idx < n_idx))
        same = em_r == en
        cnt = cnt + jnp.sum((ahead & same).astype(jnp.int32), axis=1,
                            keepdims=True)
    valid = cnt < _CAP
    flat_ref[...] = jnp.where(valid, en * _CAP + cnt, _NEG)
    iota_e = jax.lax.broadcasted_iota(jnp.int32, (blk, _E), 1)
    blk_cnt = jnp.sum((en == iota_e).astype(jnp.float32), axis=0,
                      keepdims=True)

    @pl.when(i == 0)
    def _init():
        cnt_ref[...] = jnp.zeros((1, _E), jnp.float32)
        aux_ref[...] = jnp.zeros((1, 1), jnp.float32)

    cnt_ref[...] += blk_cnt

    @pl.when(i == pl.num_programs(0) - 1)
    def _fin():
        kept = jnp.minimum(cnt_ref[...], float(_CAP))
        aux_ref[...] = (jnp.sum(kept * pm_ref[...], keepdims=True)
                        * (float(_E) / float(_N)))


def _expand_body(flat_ref, disp_ref, comb_ref):
    blk = flat_ref.shape[0]
    col = jax.lax.broadcasted_iota(jnp.int32, (blk, _E * _CAP), 1)
    d = col == flat_ref[...]
    disp_ref[...] = d
    comb_ref[...] = d.astype(jnp.float32)


def kernel(X, Wg):
    N, C, H, W = X.shape
    xr = X.reshape(N, C)
    noise = jax.random.uniform(jax.random.key(42), (N, _E),
                               dtype=jnp.float32) * _NOISE_STD

    lg, ep, ei, pm, z, std = pl.pallas_call(
        _compute_body,
        out_shape=(
            jax.ShapeDtypeStruct((N, _E), jnp.float32),
            jax.ShapeDtypeStruct((N, 1), jnp.float32),
            jax.ShapeDtypeStruct((N, 1), jnp.int32),
            jax.ShapeDtypeStruct((1, _E), jnp.float32),
            jax.ShapeDtypeStruct((1, 1), jnp.float32),
            jax.ShapeDtypeStruct((1, 1), jnp.float32),
        ),
    )(xr, Wg, noise)

    ep_row = ep.reshape(1, N)
    ei_row = ei.reshape(1, N)

    rblk = 512
    flat, _cnt, aux = pl.pallas_call(
        _rank_body,
        grid=(N // rblk,),
        in_specs=[
            pl.BlockSpec((rblk, 1), lambda i: (i, 0)),
            pl.BlockSpec((rblk, 1), lambda i: (i, 0)),
            pl.BlockSpec((1, N), lambda i: (0, 0)),
            pl.BlockSpec((1, N), lambda i: (0, 0)),
            pl.BlockSpec((1, _E), lambda i: (0, 0)),
        ],
        out_specs=(
            pl.BlockSpec((rblk, 1), lambda i: (i, 0)),
            pl.BlockSpec((1, _E), lambda i: (0, 0)),
            pl.BlockSpec((1, 1), lambda i: (0, 0)),
        ),
        out_shape=(
            jax.ShapeDtypeStruct((N, 1), jnp.int32),
            jax.ShapeDtypeStruct((1, _E), jnp.float32),
            jax.ShapeDtypeStruct((1, 1), jnp.float32),
        ),
    )(ep, ei, ep_row, ei_row, pm)

    eblk = 256
    dispatch, combine = pl.pallas_call(
        _expand_body,
        grid=(N // eblk,),
        in_specs=[pl.BlockSpec((eblk, 1), lambda i: (i, 0))],
        out_specs=(
            pl.BlockSpec((eblk, _E * _CAP), lambda i: (i, 0)),
            pl.BlockSpec((eblk, _E * _CAP), lambda i: (i, 0)),
        ),
        out_shape=(
            jax.ShapeDtypeStruct((N, _E * _CAP), jnp.bool_),
            jax.ShapeDtypeStruct((N, _E * _CAP), jnp.float32),
        ),
    )(flat)

    return (dispatch.reshape(N, _E, _CAP), combine.reshape(N, _E, _CAP),
            z[0, 0], aux[0, 0], std[0, 0], lg)


# speedup vs baseline: 1.5607x; 1.5607x over previous
import jax
import jax.numpy as jnp
from jax.experimental import pallas as pl
from jax.experimental.pallas import tpu as pltpu

_N = 4096
_E = 8
_CAP = 640
_TEMP = 1.5
_NOISE_STD = 0.02
_NEG = -1


def _compute_body(x_ref, wg_ref, noise_ref,
                  lg_ref, ep_ref, ei_ref, pm_ref, z_ref, std_ref):
    x = x_ref[...]
    wg = wg_ref[...]
    logits = jnp.dot(x, wg, preferred_element_type=jnp.float32)
    lt = jnp.clip(logits / _TEMP, -10.0, 10.0)
    m = jnp.mean(lt)
    std_ref[...] = jnp.sqrt(jnp.mean((lt - m) ** 2, keepdims=True))
    row_max = jnp.max(lt, axis=1, keepdims=True)
    lse = row_max + jnp.log(jnp.sum(jnp.exp(lt - row_max), axis=1,
                                    keepdims=True))
    z_ref[...] = jnp.mean(lse * lse, keepdims=True)
    lg = lt + noise_ref[...]
    lg_ref[...] = lg
    mx = jnp.max(lg, axis=1, keepdims=True)
    unnorm = jnp.exp(lg - mx)
    s = jnp.sum(unnorm, axis=1, keepdims=True)
    probs = unnorm / s
    pm_ref[...] = jnp.mean(probs, axis=0, keepdims=True)
    epmax = jnp.max(probs, axis=1, keepdims=True)
    ep_ref[...] = epmax
    iota_e = jax.lax.broadcasted_iota(jnp.int32, (_N, _E), 1)
    ei_ref[...] = jnp.min(jnp.where(probs == epmax, iota_e, _E), axis=1,
                          keepdims=True)


def _rank_body(ep_col_ref, ei_col_ref, ep_row_ref, ei_row_ref, pm_ref,
               flat_ref, cnt_ref, aux_ref):
    i = pl.program_id(0)
    blk = ep_col_ref.shape[0]
    pn = ep_col_ref[...]
    en = ei_col_ref[...]
    n_idx = i * blk + jax.lax.broadcasted_iota(jnp.int32, (blk, 1), 0)
    cnt = jnp.zeros((blk, 1), jnp.int32)
    chunk = 1024
    for k in range(_N // chunk):
        pm_r = ep_row_ref[:, k * chunk:(k + 1) * chunk]
        em_r = ei_row_ref[:, k * chunk:(k + 1) * chunk]
        m_idx = k * chunk + jax.lax.broadcasted_iota(
            jnp.int32, (1, chunk), 1)
        ahead = (pm_r > pn) | ((pm_r == pn) & (m_idx < n_idx))
        same = em_r == en
        cnt = cnt + jnp.sum((ahead & same).astype(jnp.int32), axis=1,
                            keepdims=True)
    valid = cnt < _CAP
    flat_ref[...] = jnp.where(valid, en * _CAP + cnt, _NEG)
    iota_e = jax.lax.broadcasted_iota(jnp.int32, (blk, _E), 1)
    blk_cnt = jnp.sum((en == iota_e).astype(jnp.float32), axis=0,
                      keepdims=True)

    @pl.when(i == 0)
    def _init():
        cnt_ref[...] = jnp.zeros((1, _E), jnp.float32)
        aux_ref[...] = jnp.zeros((1, 1), jnp.float32)

    cnt_ref[...] += blk_cnt

    @pl.when(i == pl.num_programs(0) - 1)
    def _fin():
        kept = jnp.minimum(cnt_ref[...], float(_CAP))
        aux_ref[...] = (jnp.sum(kept * pm_ref[...], keepdims=True)
                        * (float(_E) / float(_N)))


def _expand_body(flat_ref, disp_ref, comb_ref):
    blk = disp_ref.shape[0]
    e_iota = jax.lax.broadcasted_iota(jnp.int32, (blk, _E, _CAP), 1)
    c_iota = jax.lax.broadcasted_iota(jnp.int32, (blk, _E, _CAP), 2)
    d = (e_iota * _CAP + c_iota) == flat_ref[...][:, :, None]
    disp_ref[...] = d
    comb_ref[...] = d.astype(jnp.float32)


def kernel(X, Wg):
    N, C, H, W = X.shape
    xr = X.reshape(N, C)
    noise = jax.random.uniform(jax.random.key(42), (N, _E),
                               dtype=jnp.float32) * _NOISE_STD

    lg, ep, ei, pm, z, std = pl.pallas_call(
        _compute_body,
        out_shape=(
            jax.ShapeDtypeStruct((N, _E), jnp.float32),
            jax.ShapeDtypeStruct((N, 1), jnp.float32),
            jax.ShapeDtypeStruct((N, 1), jnp.int32),
            jax.ShapeDtypeStruct((1, _E), jnp.float32),
            jax.ShapeDtypeStruct((1, 1), jnp.float32),
            jax.ShapeDtypeStruct((1, 1), jnp.float32),
        ),
    )(xr, Wg, noise)

    ep_row = ep.reshape(1, N)
    ei_row = ei.reshape(1, N)

    rblk = 512
    flat, _cnt, aux = pl.pallas_call(
        _rank_body,
        grid=(N // rblk,),
        in_specs=[
            pl.BlockSpec((rblk, 1), lambda i: (i, 0)),
            pl.BlockSpec((rblk, 1), lambda i: (i, 0)),
            pl.BlockSpec((1, N), lambda i: (0, 0)),
            pl.BlockSpec((1, N), lambda i: (0, 0)),
            pl.BlockSpec((1, _E), lambda i: (0, 0)),
        ],
        out_specs=(
            pl.BlockSpec((rblk, 1), lambda i: (i, 0)),
            pl.BlockSpec((1, _E), lambda i: (0, 0)),
            pl.BlockSpec((1, 1), lambda i: (0, 0)),
        ),
        out_shape=(
            jax.ShapeDtypeStruct((N, 1), jnp.int32),
            jax.ShapeDtypeStruct((1, _E), jnp.float32),
            jax.ShapeDtypeStruct((1, 1), jnp.float32),
        ),
    )(ep, ei, ep_row, ei_row, pm)

    eblk = 256
    dispatch, combine = pl.pallas_call(
        _expand_body,
        grid=(N // eblk,),
        in_specs=[pl.BlockSpec((eblk, 1), lambda i: (i, 0))],
        out_specs=(
            pl.BlockSpec((eblk, _E, _CAP), lambda i: (i, 0, 0)),
            pl.BlockSpec((eblk, _E, _CAP), lambda i: (i, 0, 0)),
        ),
        out_shape=(
            jax.ShapeDtypeStruct((N, _E, _CAP), jnp.bool_),
            jax.ShapeDtypeStruct((N, _E, _CAP), jnp.float32),
        ),
    )(flat)

    return (dispatch, combine, z[0, 0], aux[0, 0], std[0, 0], lg)


# speedup vs baseline: 2.2698x; 1.4544x over previous
import jax
import jax.numpy as jnp
from jax.experimental import pallas as pl
from jax.experimental.pallas import tpu as pltpu

_N = 4096
_E = 8
_CAP = 640
_TEMP = 1.5
_NOISE_STD = 0.02
_KBASE = 0x3E000000
_ESHIFT = 25


def _compute_body(x_ref, wg_ref, noise_ref,
                  lg_ref, k_ref, krow_ref, suf_ref, z_ref, aux_ref, std_ref):
    x = x_ref[...]
    wg = wg_ref[...]
    logits = jnp.dot(x, wg, preferred_element_type=jnp.float32)
    lt = jnp.clip(logits / _TEMP, -10.0, 10.0)
    m = jnp.mean(lt)
    std_ref[...] = jnp.sqrt(jnp.mean((lt - m) ** 2, keepdims=True))
    row_max = jnp.max(lt, axis=1, keepdims=True)
    lse = row_max + jnp.log(jnp.sum(jnp.exp(lt - row_max), axis=1,
                                    keepdims=True))
    z_ref[...] = jnp.mean(lse * lse, keepdims=True)
    lg = lt + noise_ref[...]
    lg_ref[...] = lg
    mx = jnp.max(lg, axis=1, keepdims=True)
    unnorm = jnp.exp(lg - mx)
    s = jnp.sum(unnorm, axis=1, keepdims=True)
    probs = unnorm / s
    pm = jnp.mean(probs, axis=0, keepdims=True)
    epmax = jnp.max(probs, axis=1, keepdims=True)
    iota_e = jax.lax.broadcasted_iota(jnp.int32, (_N, _E), 1)
    ei = jnp.min(jnp.where(probs == epmax, iota_e, _E), axis=1,
                 keepdims=True)
    pbits = jax.lax.bitcast_convert_type(epmax, jnp.int32)
    kcol = (ei << _ESHIFT) | (pbits - _KBASE)
    k_ref[...] = kcol
    krow_ref[...] = jnp.transpose(kcol, (1, 0))
    cnt = jnp.sum((ei == iota_e).astype(jnp.float32), axis=0,
                  keepdims=True)
    kept = jnp.minimum(cnt, float(_CAP))
    aux_ref[...] = jnp.sum(kept * pm, keepdims=True) * (float(_E) / float(_N))
    suf_ref[...] = cnt.astype(jnp.int32)


def _dispatch_body(k_col_ref, k_row_ref, suf_ref, disp_ref, comb_ref):
    i = pl.program_id(0)
    blk = k_col_ref.shape[0]
    kn = k_col_ref[...]
    n_idx = i * blk + jax.lax.broadcasted_iota(jnp.int32, (blk, 1), 0)
    rank = jnp.zeros((blk, 1), jnp.int32)
    chunk = 1024
    for c in range(_N // chunk):
        km = k_row_ref[:, c * chunk:(c + 1) * chunk]
        m_idx = c * chunk + jax.lax.broadcasted_iota(
            jnp.int32, (1, chunk), 1)
        ahead = (km > kn) | ((km == kn) & (m_idx < n_idx))
        rank = rank + jnp.sum(ahead.astype(jnp.int32), axis=1, keepdims=True)
    en = kn >> _ESHIFT
    iota_e2 = jax.lax.broadcasted_iota(jnp.int32, (blk, _E), 1)
    sel = jnp.sum(jnp.where(iota_e2 > en, suf_ref[...], 0),
                  axis=1, keepdims=True)
    pos = rank - sel
    flat = jnp.where(pos < _CAP, en * _CAP + pos, -1)
    e_iota = jax.lax.broadcasted_iota(jnp.int32, (blk, _E, _CAP), 1)
    c_iota = jax.lax.broadcasted_iota(jnp.int32, (blk, _E, _CAP), 2)
    d = (e_iota * _CAP + c_iota) == flat[:, :, None]
    disp_ref[...] = d
    comb_ref[...] = d.astype(jnp.float32)


def kernel(X, Wg):
    N, C, H, W = X.shape
    xr = X.reshape(N, C)
    noise = jax.random.uniform(jax.random.key(42), (N, _E),
                               dtype=jnp.float32) * _NOISE_STD

    lg, k, k_row, suf, z, aux, std = pl.pallas_call(
        _compute_body,
        out_shape=(
            jax.ShapeDtypeStruct((N, _E), jnp.float32),
            jax.ShapeDtypeStruct((N, 1), jnp.int32),
            jax.ShapeDtypeStruct((1, N), jnp.int32),
            jax.ShapeDtypeStruct((1, _E), jnp.int32),
            jax.ShapeDtypeStruct((1, 1), jnp.float32),
            jax.ShapeDtypeStruct((1, 1), jnp.float32),
            jax.ShapeDtypeStruct((1, 1), jnp.float32),
        ),
    )(xr, Wg, noise)

    eblk = 256
    dispatch, combine = pl.pallas_call(
        _dispatch_body,
        grid=(N // eblk,),
        in_specs=[
            pl.BlockSpec((eblk, 1), lambda i: (i, 0)),
            pl.BlockSpec((1, N), lambda i: (0, 0)),
            pl.BlockSpec((1, _E), lambda i: (0, 0)),
        ],
        out_specs=(
            pl.BlockSpec((eblk, _E, _CAP), lambda i: (i, 0, 0)),
            pl.BlockSpec((eblk, _E, _CAP), lambda i: (i, 0, 0)),
        ),
        out_shape=(
            jax.ShapeDtypeStruct((N, _E, _CAP), jnp.bool_),
            jax.ShapeDtypeStruct((N, _E, _CAP), jnp.float32),
        ),
    )(k, k_row, suf)

    return (dispatch, combine, z[0, 0], aux[0, 0], std[0, 0], lg)


# speedup vs baseline: 2.3341x; 1.0283x over previous
import jax
import jax.numpy as jnp
from jax.experimental import pallas as pl
from jax.experimental.pallas import tpu as pltpu

_N = 4096
_E = 8
_CAP = 640
_TEMP = 1.5
_NOISE_STD = 0.02
_KBASE = 0x3E000000
_ESHIFT = 25


def _compute_body(x_ref, wg_ref, noise_ref,
                  lg_ref, k_ref, krow_ref, suf_ref, z_ref, aux_ref, std_ref):
    x = x_ref[...]
    wg = wg_ref[...]
    logits = jnp.dot(x, wg, preferred_element_type=jnp.float32)
    lt = jnp.clip(logits / _TEMP, -10.0, 10.0)
    m = jnp.mean(lt)
    std_ref[...] = jnp.sqrt(jnp.mean((lt - m) ** 2, keepdims=True))
    row_max = jnp.max(lt, axis=1, keepdims=True)
    lse = row_max + jnp.log(jnp.sum(jnp.exp(lt - row_max), axis=1,
                                    keepdims=True))
    z_ref[...] = jnp.mean(lse * lse, keepdims=True)
    lg = lt + noise_ref[...]
    lg_ref[...] = lg
    mx = jnp.max(lg, axis=1, keepdims=True)
    unnorm = jnp.exp(lg - mx)
    s = jnp.sum(unnorm, axis=1, keepdims=True)
    probs = unnorm / s
    pm = jnp.mean(probs, axis=0, keepdims=True)
    epmax = jnp.max(probs, axis=1, keepdims=True)
    iota_e = jax.lax.broadcasted_iota(jnp.int32, (_N, _E), 1)
    ei = jnp.min(jnp.where(probs == epmax, iota_e, _E), axis=1,
                 keepdims=True)
    pbits = jax.lax.bitcast_convert_type(epmax, jnp.int32)
    kcol = (ei << _ESHIFT) | (pbits - _KBASE)
    k_ref[...] = kcol
    krow_ref[...] = jnp.transpose(kcol, (1, 0))
    cnt = jnp.sum((ei == iota_e).astype(jnp.float32), axis=0,
                  keepdims=True)
    kept = jnp.minimum(cnt, float(_CAP))
    aux_ref[...] = jnp.sum(kept * pm, keepdims=True) * (float(_E) / float(_N))
    suf_ref[...] = cnt.astype(jnp.int32)


def _dispatch_body(k_col_ref, k_row_ref, suf_ref, disp_ref, comb_ref):
    i = pl.program_id(0)
    blk = k_col_ref.shape[0]
    kn = k_col_ref[...]
    n_idx = i * blk + jax.lax.broadcasted_iota(jnp.int32, (blk, 1), 0)
    rank = jnp.zeros((blk, 1), jnp.int32)
    chunk = 1024
    for c in range(_N // chunk):
        km = k_row_ref[:, c * chunk:(c + 1) * chunk]
        m_idx = c * chunk + jax.lax.broadcasted_iota(
            jnp.int32, (1, chunk), 1)
        ahead = (km > kn) | ((km == kn) & (m_idx < n_idx))
        rank = rank + jnp.sum(ahead.astype(jnp.int32), axis=1, keepdims=True)
    en = kn >> _ESHIFT
    iota_e2 = jax.lax.broadcasted_iota(jnp.int32, (blk, _E), 1)
    sel = jnp.sum(jnp.where(iota_e2 > en, suf_ref[...], 0),
                  axis=1, keepdims=True)
    pos = rank - sel
    flat = jnp.where(pos < _CAP, en * _CAP + pos, -1)
    e_iota = jax.lax.broadcasted_iota(jnp.int32, (blk, _E, _CAP), 1)
    c_iota = jax.lax.broadcasted_iota(jnp.int32, (blk, _E, _CAP), 2)
    d = (e_iota * _CAP + c_iota) == flat[:, :, None]
    disp_ref[...] = d.astype(jnp.int8)
    comb_ref[...] = d.astype(jnp.float32)


def kernel(X, Wg):
    N, C, H, W = X.shape
    xr = X.reshape(N, C)
    noise = jax.random.uniform(jax.random.key(42), (N, _E),
                               dtype=jnp.float32) * _NOISE_STD

    lg, k, k_row, suf, z, aux, std = pl.pallas_call(
        _compute_body,
        out_shape=(
            jax.ShapeDtypeStruct((N, _E), jnp.float32),
            jax.ShapeDtypeStruct((N, 1), jnp.int32),
            jax.ShapeDtypeStruct((1, N), jnp.int32),
            jax.ShapeDtypeStruct((1, _E), jnp.int32),
            jax.ShapeDtypeStruct((1, 1), jnp.float32),
            jax.ShapeDtypeStruct((1, 1), jnp.float32),
            jax.ShapeDtypeStruct((1, 1), jnp.float32),
        ),
    )(xr, Wg, noise)

    eblk = 256
    dispatch, combine = pl.pallas_call(
        _dispatch_body,
        grid=(N // eblk,),
        in_specs=[
            pl.BlockSpec((eblk, 1), lambda i: (i, 0)),
            pl.BlockSpec((1, N), lambda i: (0, 0)),
            pl.BlockSpec((1, _E), lambda i: (0, 0)),
        ],
        out_specs=(
            pl.BlockSpec((eblk, _E, _CAP), lambda i: (i, 0, 0)),
            pl.BlockSpec((eblk, _E, _CAP), lambda i: (i, 0, 0)),
        ),
        out_shape=(
            jax.ShapeDtypeStruct((N, _E, _CAP), jnp.int8),
            jax.ShapeDtypeStruct((N, _E, _CAP), jnp.float32),
        ),
    )(k, k_row, suf)

    dispatch = dispatch.view(jnp.bool_)
    return (dispatch, combine, z[0, 0], aux[0, 0], std[0, 0], lg)


# speedup vs baseline: 2.3506x; 1.0071x over previous
import jax
import jax.numpy as jnp
from jax.experimental import pallas as pl
from jax.experimental.pallas import tpu as pltpu

_N = 4096
_E = 8
_CAP = 640
_TEMP = 1.5
_NOISE_STD = 0.02
_KBASE = 0x3E000000
_ESHIFT = 25
_EBLK = 256


def _fused_body(x_ref, wg_ref, noise_ref,
                lg_ref, z_ref, aux_ref, std_ref, disp_ref, comb_ref,
                kcol_s, krow_s, suf_s):
    i = pl.program_id(0)

    @pl.when(i == 0)
    def _compute():
        x = x_ref[...]
        wg = wg_ref[...]
        logits = jnp.dot(x, wg, preferred_element_type=jnp.float32)
        lt = jnp.clip(logits / _TEMP, -10.0, 10.0)
        m = jnp.mean(lt)
        std_ref[...] = jnp.sqrt(jnp.mean((lt - m) ** 2, keepdims=True))
        row_max = jnp.max(lt, axis=1, keepdims=True)
        lse = row_max + jnp.log(jnp.sum(jnp.exp(lt - row_max), axis=1,
                                        keepdims=True))
        z_ref[...] = jnp.mean(lse * lse, keepdims=True)
        lg = lt + noise_ref[...]
        lg_ref[...] = lg
        mx = jnp.max(lg, axis=1, keepdims=True)
        unnorm = jnp.exp(lg - mx)
        s = jnp.sum(unnorm, axis=1, keepdims=True)
        probs = unnorm / s
        pm = jnp.mean(probs, axis=0, keepdims=True)
        epmax = jnp.max(probs, axis=1, keepdims=True)
        iota_e = jax.lax.broadcasted_iota(jnp.int32, (_N, _E), 1)
        ei = jnp.min(jnp.where(probs == epmax, iota_e, _E), axis=1,
                     keepdims=True)
        pbits = jax.lax.bitcast_convert_type(epmax, jnp.int32)
        kcol = (ei << _ESHIFT) | (pbits - _KBASE)
        kcol_s[...] = kcol
        krow_s[...] = jnp.transpose(kcol, (1, 0))
        cnt = jnp.sum((ei == iota_e).astype(jnp.float32), axis=0,
                      keepdims=True)
        kept = jnp.minimum(cnt, float(_CAP))
        aux_ref[...] = (jnp.sum(kept * pm, keepdims=True)
                        * (float(_E) / float(_N)))
        suf_s[...] = cnt.astype(jnp.int32)

    @pl.when(i > 0)
    def _disp():
        j = i - 1
        kn = kcol_s[pl.ds(j * _EBLK, _EBLK), :]
        n_idx = (j * _EBLK
                 + jax.lax.broadcasted_iota(jnp.int32, (_EBLK, 1), 0))
        rank = jnp.zeros((_EBLK, 1), jnp.int32)
        chunk = 1024
        for c in range(_N // chunk):
            km = krow_s[:, c * chunk:(c + 1) * chunk]
            m_idx = c * chunk + jax.lax.broadcasted_iota(
                jnp.int32, (1, chunk), 1)
            ahead = (km > kn) | ((km == kn) & (m_idx < n_idx))
            rank = rank + jnp.sum(ahead.astype(jnp.int32), axis=1,
                                  keepdims=True)
        en = kn >> _ESHIFT
        iota_e2 = jax.lax.broadcasted_iota(jnp.int32, (_EBLK, _E), 1)
        sel = jnp.sum(jnp.where(iota_e2 > en, suf_s[...], 0),
                      axis=1, keepdims=True)
        pos = rank - sel
        flat = jnp.where(pos < _CAP, en * _CAP + pos, -1)
        e_iota = jax.lax.broadcasted_iota(jnp.int32, (_EBLK, _E, _CAP), 1)
        c_iota = jax.lax.broadcasted_iota(jnp.int32, (_EBLK, _E, _CAP), 2)
        d = (e_iota * _CAP + c_iota) == flat[:, :, None]
        disp_ref[...] = d.astype(jnp.int8)
        comb_ref[...] = d.astype(jnp.float32)


def kernel(X, Wg):
    N, C, H, W = X.shape
    xr = X.reshape(N, C)
    noise = jax.random.uniform(jax.random.key(42), (N, _E),
                               dtype=jnp.float32) * _NOISE_STD

    zero = lambda i: (0, 0)
    lg, z, aux, std, dispatch, combine = pl.pallas_call(
        _fused_body,
        grid=(1 + N // _EBLK,),
        in_specs=[
            pl.BlockSpec((N, C), zero),
            pl.BlockSpec((C, _E), zero),
            pl.BlockSpec((N, _E), zero),
        ],
        out_specs=(
            pl.BlockSpec((N, _E), zero),
            pl.BlockSpec((1, 1), zero),
            pl.BlockSpec((1, 1), zero),
            pl.BlockSpec((1, 1), zero),
            pl.BlockSpec((_EBLK, _E, _CAP),
                         lambda i: (jnp.maximum(i - 1, 0), 0, 0)),
            pl.BlockSpec((_EBLK, _E, _CAP),
                         lambda i: (jnp.maximum(i - 1, 0), 0, 0)),
        ),
        out_shape=(
            jax.ShapeDtypeStruct((N, _E), jnp.float32),
            jax.ShapeDtypeStruct((1, 1), jnp.float32),
            jax.ShapeDtypeStruct((1, 1), jnp.float32),
            jax.ShapeDtypeStruct((1, 1), jnp.float32),
            jax.ShapeDtypeStruct((N, _E, _CAP), jnp.int8),
            jax.ShapeDtypeStruct((N, _E, _CAP), jnp.float32),
        ),
        scratch_shapes=[
            pltpu.VMEM((_N, 1), jnp.int32),
            pltpu.VMEM((1, _N), jnp.int32),
            pltpu.VMEM((1, _E), jnp.int32),
        ],
    )(xr, Wg, noise)

    dispatch = dispatch.view(jnp.bool_)
    return (dispatch, combine, z[0, 0], aux[0, 0], std[0, 0], lg)


# speedup vs baseline: 2.3532x; 1.0011x over previous
import jax
import jax.numpy as jnp
from jax.experimental import pallas as pl
from jax.experimental.pallas import tpu as pltpu

_N = 4096
_E = 8
_CAP = 640
_TEMP = 1.5
_NOISE_STD = 0.02
_KBASE = 0x3E000000
_ESHIFT = 25
_EBLK = 512


def _fused_body(x_ref, wg_ref, noise_ref,
                lg_ref, z_ref, aux_ref, std_ref, disp_ref, comb_ref,
                kcol_s, krow_s, suf_s):
    i = pl.program_id(0)

    @pl.when(i == 0)
    def _compute():
        x = x_ref[...]
        wg = wg_ref[...]
        logits = jnp.dot(x, wg, preferred_element_type=jnp.float32)
        lt = jnp.clip(logits / _TEMP, -10.0, 10.0)
        m = jnp.mean(lt)
        std_ref[...] = jnp.sqrt(jnp.mean((lt - m) ** 2, keepdims=True))
        row_max = jnp.max(lt, axis=1, keepdims=True)
        lse = row_max + jnp.log(jnp.sum(jnp.exp(lt - row_max), axis=1,
                                        keepdims=True))
        z_ref[...] = jnp.mean(lse * lse, keepdims=True)
        lg = lt + noise_ref[...]
        lg_ref[...] = lg
        mx = jnp.max(lg, axis=1, keepdims=True)
        unnorm = jnp.exp(lg - mx)
        s = jnp.sum(unnorm, axis=1, keepdims=True)
        probs = unnorm / s
        pm = jnp.mean(probs, axis=0, keepdims=True)
        epmax = jnp.max(probs, axis=1, keepdims=True)
        iota_e = jax.lax.broadcasted_iota(jnp.int32, (_N, _E), 1)
        ei = jnp.min(jnp.where(probs == epmax, iota_e, _E), axis=1,
                     keepdims=True)
        pbits = jax.lax.bitcast_convert_type(epmax, jnp.int32)
        kcol = (ei << _ESHIFT) | (pbits - _KBASE)
        kcol_s[...] = kcol
        krow_s[...] = jnp.transpose(kcol, (1, 0))
        cnt = jnp.sum((ei == iota_e).astype(jnp.float32), axis=0,
                      keepdims=True)
        kept = jnp.minimum(cnt, float(_CAP))
        aux_ref[...] = (jnp.sum(kept * pm, keepdims=True)
                        * (float(_E) / float(_N)))
        suf_s[...] = cnt.astype(jnp.int32)

    @pl.when(i > 0)
    def _disp():
        j = i - 1
        kn = kcol_s[pl.ds(j * _EBLK, _EBLK), :]
        n_idx = (j * _EBLK
                 + jax.lax.broadcasted_iota(jnp.int32, (_EBLK, 1), 0))
        rank = jnp.zeros((_EBLK, 1), jnp.int32)
        chunk = 1024
        for c in range(_N // chunk):
            km = krow_s[:, c * chunk:(c + 1) * chunk]
            m_idx = c * chunk + jax.lax.broadcasted_iota(
                jnp.int32, (1, chunk), 1)
            ahead = (km > kn) | ((km == kn) & (m_idx < n_idx))
            rank = rank + jnp.sum(ahead.astype(jnp.int32), axis=1,
                                  keepdims=True)
        en = kn >> _ESHIFT
        iota_e2 = jax.lax.broadcasted_iota(jnp.int32, (_EBLK, _E), 1)
        sel = jnp.sum(jnp.where(iota_e2 > en, suf_s[...], 0),
                      axis=1, keepdims=True)
        pos = rank - sel
        flat = jnp.where(pos < _CAP, en * _CAP + pos, -1)
        e_iota = jax.lax.broadcasted_iota(jnp.int32, (_EBLK, _E, _CAP), 1)
        c_iota = jax.lax.broadcasted_iota(jnp.int32, (_EBLK, _E, _CAP), 2)
        d = (e_iota * _CAP + c_iota) == flat[:, :, None]
        disp_ref[...] = d.astype(jnp.int8)
        comb_ref[...] = d.astype(jnp.float32)


def kernel(X, Wg):
    N, C, H, W = X.shape
    xr = X.reshape(N, C)
    noise = jax.random.uniform(jax.random.key(42), (N, _E),
                               dtype=jnp.float32) * _NOISE_STD

    zero = lambda i: (0, 0)
    lg, z, aux, std, dispatch, combine = pl.pallas_call(
        _fused_body,
        grid=(1 + N // _EBLK,),
        in_specs=[
            pl.BlockSpec((N, C), zero),
            pl.BlockSpec((C, _E), zero),
            pl.BlockSpec((N, _E), zero),
        ],
        out_specs=(
            pl.BlockSpec((N, _E), zero),
            pl.BlockSpec((1, 1), zero),
            pl.BlockSpec((1, 1), zero),
            pl.BlockSpec((1, 1), zero),
            pl.BlockSpec((_EBLK, _E, _CAP),
                         lambda i: (jnp.maximum(i - 1, 0), 0, 0)),
            pl.BlockSpec((_EBLK, _E, _CAP),
                         lambda i: (jnp.maximum(i - 1, 0), 0, 0)),
        ),
        out_shape=(
            jax.ShapeDtypeStruct((N, _E), jnp.float32),
            jax.ShapeDtypeStruct((1, 1), jnp.float32),
            jax.ShapeDtypeStruct((1, 1), jnp.float32),
            jax.ShapeDtypeStruct((1, 1), jnp.float32),
            jax.ShapeDtypeStruct((N, _E, _CAP), jnp.int8),
            jax.ShapeDtypeStruct((N, _E, _CAP), jnp.float32),
        ),
        scratch_shapes=[
            pltpu.VMEM((_N, 1), jnp.int32),
            pltpu.VMEM((1, _N), jnp.int32),
            pltpu.VMEM((1, _E), jnp.int32),
        ],
    )(xr, Wg, noise)

    dispatch = dispatch.view(jnp.bool_)
    return (dispatch, combine, z[0, 0], aux[0, 0], std[0, 0], lg)
